# Initial kernel scaffold; baseline (speedup 1.0000x reference)
import jax
import jax.numpy as jnp
from jax.experimental import pallas as pl


def kernel(x, edge_index, W0, b0, Wr0, W1, b1, Wr1):
    raise NotImplementedError("write your pallas kernel here")


# speedup vs baseline: 7.9551x; 7.9551x over previous
import functools

import jax
import jax.numpy as jnp
from jax import lax
from jax.experimental import pallas as pl
from jax.experimental.pallas import tpu as pltpu
from jax.experimental.pallas import tpu_sc as plsc

N = 10000
H = 128
FW = 64
E = 320000
NC = 2
NS = 16
C = 80
EW = E // NS
KC = EW // C
RPT = N // NS
ZR = 25


def _make_scatter():
    mesh = plsc.VectorSubcoreMesh(core_axis_name="c", subcore_axis_name="s")

    @functools.partial(
        pl.kernel,
        out_type=jax.ShapeDtypeStruct((NC, NS, RPT, FW), jnp.float32),
        mesh=mesh,
        scratch_types=[
            pltpu.VMEM((KC, C), jnp.int32),
            pltpu.VMEM((KC, C), jnp.int32),
            pltpu.VMEM((C, FW), jnp.float32),
            pltpu.VMEM((C, FW), jnp.float32),
            pltpu.VMEM((ZR, FW), jnp.float32),
            pltpu.VMEM_SHARED((N, FW), jnp.float32),
            pltpu.SemaphoreType.DMA,
            pltpu.SemaphoreType.DMA,
        ],
        compiler_params=pltpu.CompilerParams(use_tc_tiling_on_sc=False),
    )
    def scatter_k(hs_hbm, src_hbm, dst_hbm, out_hbm,
                  src_v, dst_v, buf0, buf1, zbuf, acc, sem0, sem1):
        cid = lax.axis_index("c")
        sid = lax.axis_index("s")
        table = hs_hbm.at[cid]

        @pl.loop(0, ZR)
        def _(i):
            @pl.loop(0, FW // 16)
            def _(j):
                zbuf[i, pl.ds(j * 16, 16)] = jnp.zeros((16,), jnp.float32)

        @pl.loop(0, RPT // ZR)
        def _(k):
            pltpu.sync_copy(zbuf, acc.at[pl.ds(sid * RPT + k * ZR, ZR)])

        plsc.subcore_barrier()

        pltpu.sync_copy(src_hbm.at[sid], src_v)
        pltpu.sync_copy(dst_hbm.at[sid], dst_v)

        pltpu.async_copy(table.at[src_v.at[0]], buf0, sem0)
        pltpu.async_copy(table.at[src_v.at[1]], buf1, sem1)

        @pl.loop(0, (KC - 2) // 2)
        def _(k):
            i0 = 2 * k
            pltpu.make_async_copy(table.at[src_v.at[i0]], buf0, sem0).wait()
            pltpu.sync_copy(buf0, acc.at[dst_v.at[i0]], add=True)
            pltpu.async_copy(table.at[src_v.at[i0 + 2]], buf0, sem0)
            pltpu.make_async_copy(table.at[src_v.at[i0 + 1]], buf1, sem1).wait()
            pltpu.sync_copy(buf1, acc.at[dst_v.at[i0 + 1]], add=True)
            pltpu.async_copy(table.at[src_v.at[i0 + 3]], buf1, sem1)

        pltpu.make_async_copy(table.at[src_v.at[KC - 2]], buf0, sem0).wait()
        pltpu.sync_copy(buf0, acc.at[dst_v.at[KC - 2]], add=True)
        pltpu.make_async_copy(table.at[src_v.at[KC - 1]], buf1, sem1).wait()
        pltpu.sync_copy(buf1, acc.at[dst_v.at[KC - 1]], add=True)

        plsc.subcore_barrier()

        pltpu.sync_copy(acc.at[pl.ds(sid * RPT, RPT)], out_hbm.at[cid, sid])

    return scatter_k


_scatter = _make_scatter()

_BN = 1000


def _combine_body(pa_ref, pb_ref, h_ref, wt_ref, wb_ref, wr_ref, b_ref, o_ref):
    acc = jnp.dot(pa_ref[...], wt_ref[...], preferred_element_type=jnp.float32)
    acc += jnp.dot(pb_ref[...], wb_ref[...], preferred_element_type=jnp.float32)
    acc += jnp.dot(h_ref[...], wr_ref[...], preferred_element_type=jnp.float32)
    o_ref[...] = jnp.maximum(acc + b_ref[...], 0.0)


def _combine(pa, pb, h, W, b2, Wr):
    return pl.pallas_call(
        _combine_body,
        grid=(N // _BN,),
        in_specs=[
            pl.BlockSpec((_BN, FW), lambda i: (i, 0)),
            pl.BlockSpec((_BN, FW), lambda i: (i, 0)),
            pl.BlockSpec((_BN, H), lambda i: (i, 0)),
            pl.BlockSpec((FW, H), lambda i: (0, 0)),
            pl.BlockSpec((FW, H), lambda i: (0, 0)),
            pl.BlockSpec((H, H), lambda i: (0, 0)),
            pl.BlockSpec((1, H), lambda i: (0, 0)),
        ],
        out_specs=pl.BlockSpec((_BN, H), lambda i: (i, 0)),
        out_shape=jax.ShapeDtypeStruct((N, H), jnp.float32),
    )(pa, pb, h, W[:FW], W[FW:], Wr, b2)


def kernel(x, edge_index, W0, b0, Wr0, W1, b1, Wr1):
    src3d = edge_index[0].reshape(NS, KC, C)
    dst3d = edge_index[1].reshape(NS, KC, C)
    b0r = b0.reshape(1, H)
    b1r = b1.reshape(1, H)

    xs = jnp.stack([x[:, :FW], x[:, FW:]])
    p0 = _scatter(xs, src3d, dst3d).reshape(NC, N, FW)
    h1 = _combine(p0[0], p0[1], x, W0, b0r, Wr0)
    hs1 = jnp.stack([h1[:, :FW], h1[:, FW:]])
    p1 = _scatter(hs1, src3d, dst3d).reshape(NC, N, FW)
    h2 = _combine(p1[0], p1[1], h1, W1, b1r, Wr1)
    return h2


# speedup vs baseline: 9.2464x; 1.1623x over previous
import functools

import jax
import jax.numpy as jnp
from jax import lax
from jax.experimental import pallas as pl
from jax.experimental.pallas import tpu as pltpu
from jax.experimental.pallas import tpu_sc as plsc

N = 10000
H = 128
FW = 64
E = 320000
NC = 2
NS = 16
C = 125
EW = E // NS
KC = EW // C
RPT = N // NS
ZR = 25


def _make_scatter():
    mesh = plsc.VectorSubcoreMesh(core_axis_name="c", subcore_axis_name="s")

    @functools.partial(
        pl.kernel,
        out_type=jax.ShapeDtypeStruct((NC, NS, RPT, FW), jnp.float32),
        mesh=mesh,
        scratch_types=[
            pltpu.VMEM((KC, C), jnp.int32),
            pltpu.VMEM((KC, C), jnp.int32),
            pltpu.VMEM((C, FW), jnp.float32),
            pltpu.VMEM((C, FW), jnp.float32),
            pltpu.VMEM((C, FW), jnp.float32),
            pltpu.VMEM((C, FW), jnp.float32),
            pltpu.VMEM((ZR, FW), jnp.float32),
            pltpu.VMEM_SHARED((N, FW), jnp.float32),
            pltpu.SemaphoreType.DMA,
            pltpu.SemaphoreType.DMA,
            pltpu.SemaphoreType.DMA,
            pltpu.SemaphoreType.DMA,
            pltpu.SemaphoreType.DMA,
            pltpu.SemaphoreType.DMA,
            pltpu.SemaphoreType.DMA,
            pltpu.SemaphoreType.DMA,
        ],
        compiler_params=pltpu.CompilerParams(use_tc_tiling_on_sc=False),
    )
    def scatter_k(hs_hbm, src_hbm, dst_hbm, out_hbm,
                  src_v, dst_v, b0, b1, b2, b3, zbuf, acc,
                  g0, g1, g2, g3, s0, s1, s2, s3):
        cid = lax.axis_index("c")
        sid = lax.axis_index("s")
        table = hs_hbm.at[cid]
        bufs = (b0, b1, b2, b3)
        gsems = (g0, g1, g2, g3)
        ssems = (s0, s1, s2, s3)

        def g_issue(i, b):
            pltpu.async_copy(table.at[src_v.at[i]], bufs[b], gsems[b])

        def g_wait(i, b):
            pltpu.make_async_copy(table.at[src_v.at[i]], bufs[b], gsems[b]).wait()

        def s_issue(i, b):
            pltpu.async_copy(bufs[b], acc.at[dst_v.at[i]], ssems[b], add=True)

        def s_wait(i, b):
            pltpu.make_async_copy(bufs[b], acc.at[dst_v.at[i]], ssems[b]).wait()

        @pl.loop(0, ZR)
        def _(i):
            @pl.loop(0, FW // 16)
            def _(j):
                zbuf[i, pl.ds(j * 16, 16)] = jnp.zeros((16,), jnp.float32)

        @pl.loop(0, RPT // ZR)
        def _(k):
            pltpu.sync_copy(zbuf, acc.at[pl.ds(sid * RPT + k * ZR, ZR)])

        plsc.subcore_barrier()

        pltpu.sync_copy(src_hbm.at[sid], src_v)
        pltpu.sync_copy(dst_hbm.at[sid], dst_v)

        g_issue(0, 0)
        g_issue(1, 1)
        g_wait(0, 0)
        s_issue(0, 0)
        g_issue(2, 2)
        g_wait(1, 1)
        s_issue(1, 1)
        g_issue(3, 3)

        @pl.loop(0, (KC - 4) // 4)
        def _(k):
            i = 4 * k
            for off in (2, 3, 4, 5):
                bb = off % 4
                g_wait(i + off, bb)
                s_issue(i + off, bb)
                s_wait(i + off - 2, (off - 2) % 4)
                g_issue(i + off + 2, (off + 2) % 4)

        g_wait(KC - 2, 2)
        s_issue(KC - 2, 2)
        s_wait(KC - 4, 0)
        g_wait(KC - 1, 3)
        s_issue(KC - 1, 3)
        s_wait(KC - 3, 1)
        s_wait(KC - 2, 2)
        s_wait(KC - 1, 3)

        plsc.subcore_barrier()

        pltpu.sync_copy(acc.at[pl.ds(sid * RPT, RPT)], out_hbm.at[cid, sid])

    return scatter_k


_scatter = _make_scatter()

_BN = 1000


def _combine_body(split_out, pa_ref, pb_ref, ha_ref, hb_ref,
                  wt_ref, wb_ref, wrt_ref, wrb_ref, b_ref, o_ref):
    acc = jnp.dot(pa_ref[...], wt_ref[...], preferred_element_type=jnp.float32)
    acc += jnp.dot(pb_ref[...], wb_ref[...], preferred_element_type=jnp.float32)
    acc += jnp.dot(ha_ref[...], wrt_ref[...], preferred_element_type=jnp.float32)
    acc += jnp.dot(hb_ref[...], wrb_ref[...], preferred_element_type=jnp.float32)
    res = jnp.maximum(acc + b_ref[...], 0.0)
    if split_out:
        o_ref[0] = res[:, :FW]
        o_ref[1] = res[:, FW:]
    else:
        o_ref[...] = res


def _combine(pa, pb, ha, hb, W, b2, Wr, split_out):
    if split_out:
        out_spec = pl.BlockSpec((NC, _BN, FW), lambda i: (0, i, 0))
        out_shape = jax.ShapeDtypeStruct((NC, N, FW), jnp.float32)
    else:
        out_spec = pl.BlockSpec((_BN, H), lambda i: (i, 0))
        out_shape = jax.ShapeDtypeStruct((N, H), jnp.float32)
    return pl.pallas_call(
        functools.partial(_combine_body, split_out),
        grid=(N // _BN,),
        in_specs=[
            pl.BlockSpec((_BN, FW), lambda i: (i, 0)),
            pl.BlockSpec((_BN, FW), lambda i: (i, 0)),
            pl.BlockSpec((_BN, FW), lambda i: (i, 0)),
            pl.BlockSpec((_BN, FW), lambda i: (i, 0)),
            pl.BlockSpec((FW, H), lambda i: (0, 0)),
            pl.BlockSpec((FW, H), lambda i: (0, 0)),
            pl.BlockSpec((FW, H), lambda i: (0, 0)),
            pl.BlockSpec((FW, H), lambda i: (0, 0)),
            pl.BlockSpec((1, H), lambda i: (0, 0)),
        ],
        out_specs=out_spec,
        out_shape=out_shape,
    )(pa, pb, ha, hb, W[:FW], W[FW:], Wr[:FW], Wr[FW:], b2)


def kernel(x, edge_index, W0, b0, Wr0, W1, b1, Wr1):
    src3d = edge_index[0].reshape(NS, KC, C)
    dst3d = edge_index[1].reshape(NS, KC, C)
    b0r = b0.reshape(1, H)
    b1r = b1.reshape(1, H)

    xs = jnp.stack([x[:, :FW], x[:, FW:]])
    p0 = _scatter(xs, src3d, dst3d).reshape(NC, N, FW)
    hs1 = _combine(p0[0], p0[1], xs[0], xs[1], W0, b0r, Wr0, split_out=True)
    p1 = _scatter(hs1, src3d, dst3d).reshape(NC, N, FW)
    h2 = _combine(p1[0], p1[1], hs1[0], hs1[1], W1, b1r, Wr1, split_out=False)
    return h2


# speedup vs baseline: 10.7429x; 1.1618x over previous
import functools

import jax
import jax.numpy as jnp
from jax import lax
from jax.experimental import pallas as pl
from jax.experimental.pallas import tpu as pltpu
from jax.experimental.pallas import tpu_sc as plsc

N = 10000
H = 128
FW = 64
E = 320000
NC = 2
NS = 16
C = 125
EW = E // NS
KC = EW // C
RPT = N // NS
ZR = 25


def _make_scatter():
    mesh = plsc.VectorSubcoreMesh(core_axis_name="c", subcore_axis_name="s")

    @functools.partial(
        pl.kernel,
        out_type=jax.ShapeDtypeStruct((NC, N, FW), jnp.float32),
        mesh=mesh,
        scratch_types=[
            pltpu.VMEM((KC, C), jnp.int32),
            pltpu.VMEM((KC, C), jnp.int32),
            pltpu.VMEM((C, FW), jnp.float32),
            pltpu.VMEM((C, FW), jnp.float32),
            pltpu.VMEM((C, FW), jnp.float32),
            pltpu.VMEM((C, FW), jnp.float32),
            pltpu.VMEM((ZR, FW), jnp.float32),
            pltpu.VMEM_SHARED((N, FW), jnp.float32),
            pltpu.SemaphoreType.DMA,
            pltpu.SemaphoreType.DMA,
            pltpu.SemaphoreType.DMA,
            pltpu.SemaphoreType.DMA,
            pltpu.SemaphoreType.DMA,
            pltpu.SemaphoreType.DMA,
            pltpu.SemaphoreType.DMA,
            pltpu.SemaphoreType.DMA,
        ],
        compiler_params=pltpu.CompilerParams(use_tc_tiling_on_sc=False),
    )
    def scatter_k(h_hbm, e_hbm, out_hbm,
                  src_v, dst_v, b0, b1, b2, b3, zbuf, acc,
                  g0, g1, g2, g3, s0, s1, s2, s3):
        cid = lax.axis_index("c")
        sid = lax.axis_index("s")
        table = h_hbm.at[cid]
        bufs = (b0, b1, b2, b3)
        gsems = (g0, g1, g2, g3)
        ssems = (s0, s1, s2, s3)

        def g_issue(i, b):
            pltpu.async_copy(table.at[src_v.at[i]], bufs[b], gsems[b])

        def g_wait(i, b):
            pltpu.make_async_copy(table.at[src_v.at[i]], bufs[b], gsems[b]).wait()

        def s_issue(i, b):
            pltpu.async_copy(bufs[b], acc.at[dst_v.at[i]], ssems[b], add=True)

        def s_wait(i, b):
            pltpu.make_async_copy(bufs[b], acc.at[dst_v.at[i]], ssems[b]).wait()

        @pl.loop(0, ZR)
        def _(i):
            @pl.loop(0, FW // 16)
            def _(j):
                zbuf[i, pl.ds(j * 16, 16)] = jnp.zeros((16,), jnp.float32)

        @pl.loop(0, RPT // ZR)
        def _(k):
            pltpu.sync_copy(zbuf, acc.at[pl.ds(sid * RPT + k * ZR, ZR)])

        plsc.subcore_barrier()

        pltpu.sync_copy(e_hbm.at[0, sid], src_v)
        pltpu.sync_copy(e_hbm.at[1, sid], dst_v)

        g_issue(0, 0)
        g_issue(1, 1)
        g_wait(0, 0)
        s_issue(0, 0)
        g_issue(2, 2)
        g_wait(1, 1)
        s_issue(1, 1)
        g_issue(3, 3)

        @pl.loop(0, (KC - 4) // 4)
        def _(k):
            i = 4 * k
            for off in (2, 3, 4, 5):
                bb = off % 4
                g_wait(i + off, bb)
                s_issue(i + off, bb)
                s_wait(i + off - 2, (off - 2) % 4)
                g_issue(i + off + 2, (off + 2) % 4)

        g_wait(KC - 2, 2)
        s_issue(KC - 2, 2)
        s_wait(KC - 4, 0)
        g_wait(KC - 1, 3)
        s_issue(KC - 1, 3)
        s_wait(KC - 3, 1)
        s_wait(KC - 2, 2)
        s_wait(KC - 1, 3)

        plsc.subcore_barrier()

        pltpu.sync_copy(acc.at[pl.ds(sid * RPT, RPT)],
                        out_hbm.at[cid, pl.ds(sid * RPT, RPT)])

    return scatter_k


_scatter = _make_scatter()

_BN = 1000


def _combine_body(split_out, p_ref, h_ref, wt_ref, wb_ref, wr_ref, b_ref, o_ref):
    acc = jnp.dot(p_ref[0], wt_ref[...], preferred_element_type=jnp.float32)
    acc += jnp.dot(p_ref[1], wb_ref[...], preferred_element_type=jnp.float32)
    if split_out:
        acc += jnp.dot(h_ref[...], wr_ref[...], preferred_element_type=jnp.float32)
    else:
        acc += jnp.dot(h_ref[0], wr_ref[:FW], preferred_element_type=jnp.float32)
        acc += jnp.dot(h_ref[1], wr_ref[FW:], preferred_element_type=jnp.float32)
    res = jnp.maximum(acc + b_ref[...], 0.0)
    if split_out:
        o_ref[0] = res[:, :FW]
        o_ref[1] = res[:, FW:]
    else:
        o_ref[...] = res


def _combine(p, h, W, b2, Wr, split_out):
    if split_out:
        h_spec = pl.BlockSpec((_BN, H), lambda i: (i, 0))
        out_spec = pl.BlockSpec((NC, _BN, FW), lambda i: (0, i, 0))
        out_shape = jax.ShapeDtypeStruct((NC, N, FW), jnp.float32)
    else:
        h_spec = pl.BlockSpec((NC, _BN, FW), lambda i: (0, i, 0))
        out_spec = pl.BlockSpec((_BN, H), lambda i: (i, 0))
        out_shape = jax.ShapeDtypeStruct((N, H), jnp.float32)
    return pl.pallas_call(
        functools.partial(_combine_body, split_out),
        grid=(N // _BN,),
        in_specs=[
            pl.BlockSpec((NC, _BN, FW), lambda i: (0, i, 0)),
            h_spec,
            pl.BlockSpec((FW, H), lambda i: (0, 0)),
            pl.BlockSpec((FW, H), lambda i: (0, 0)),
            pl.BlockSpec((H, H), lambda i: (0, 0)),
            pl.BlockSpec((1, H), lambda i: (0, 0)),
        ],
        out_specs=out_spec,
        out_shape=out_shape,
    )(p, h, W[:FW], W[FW:], Wr, b2)


def kernel(x, edge_index, W0, b0, Wr0, W1, b1, Wr1):
    e4d = edge_index.reshape(2, NS, KC, C)
    b0r = b0.reshape(1, H)
    b1r = b1.reshape(1, H)

    xs = jnp.stack([x[:, :FW], x[:, FW:]])
    p0 = _scatter(xs, e4d)
    hs1 = _combine(p0, x, W0, b0r, Wr0, split_out=True)
    p1 = _scatter(hs1, e4d)
    h2 = _combine(p1, hs1, W1, b1r, Wr1, split_out=False)
    return h2


# speedup vs baseline: 10.7601x; 1.0016x over previous
import functools

import jax
import jax.numpy as jnp
from jax import lax
from jax.experimental import pallas as pl
from jax.experimental.pallas import tpu as pltpu
from jax.experimental.pallas import tpu_sc as plsc

N = 10000
H = 128
FW = 64
E = 320000
NC = 2
NS = 16
C = 80
EW = E // NS
KC = EW // C
RPT = N // NS
ZR = 25


def _make_scatter():
    mesh = plsc.VectorSubcoreMesh(core_axis_name="c", subcore_axis_name="s")

    @functools.partial(
        pl.kernel,
        out_type=(
            jax.ShapeDtypeStruct((N, H), jnp.float32),
            jax.ShapeDtypeStruct((NC, N, FW), jnp.float32),
        ),
        mesh=mesh,
        scratch_types=[
            pltpu.VMEM((EW,), jnp.int32),
            pltpu.VMEM((EW,), jnp.int32),
            pltpu.VMEM((C, FW), jnp.float32),
            pltpu.VMEM((C, FW), jnp.float32),
            pltpu.VMEM((C, FW), jnp.float32),
            pltpu.VMEM((C, FW), jnp.float32),
            pltpu.VMEM((ZR, FW), jnp.float32),
            pltpu.VMEM_SHARED((N, FW), jnp.float32),
            pltpu.SemaphoreType.DMA,
            pltpu.SemaphoreType.DMA,
            pltpu.SemaphoreType.DMA,
            pltpu.SemaphoreType.DMA,
            pltpu.SemaphoreType.DMA,
            pltpu.SemaphoreType.DMA,
            pltpu.SemaphoreType.DMA,
            pltpu.SemaphoreType.DMA,
        ],
        compiler_params=pltpu.CompilerParams(use_tc_tiling_on_sc=False),
    )
    def scatter_k(h_hbm, e_hbm, out_hbm, tab_hbm,
                  src_v, dst_v, b0, b1, b2, b3, zbuf, acc,
                  g0, g1, g2, g3, s0, s1, s2, s3):
        cid = lax.axis_index("c")
        sid = lax.axis_index("s")
        table = tab_hbm.at[cid]
        bufs = (b0, b1, b2, b3)
        gsems = (g0, g1, g2, g3)
        ssems = (s0, s1, s2, s3)

        def src_idx(i):
            return src_v.at[pl.ds(i * C, C)]

        def dst_idx(i):
            return dst_v.at[pl.ds(i * C, C)]

        def g_issue(i, b):
            pltpu.async_copy(table.at[src_idx(i)], bufs[b], gsems[b])

        def g_wait(i, b):
            pltpu.make_async_copy(table.at[src_idx(i)], bufs[b], gsems[b]).wait()

        def s_issue(i, b):
            pltpu.async_copy(bufs[b], acc.at[dst_idx(i)], ssems[b], add=True)

        def s_wait(i, b):
            pltpu.make_async_copy(bufs[b], acc.at[dst_idx(i)], ssems[b]).wait()

        @pl.loop(0, RPT // C)
        def _(j):
            r0 = sid * RPT + j * C
            pltpu.sync_copy(h_hbm.at[pl.ds(r0, C), pl.ds(cid * FW, FW)], b0)
            pltpu.sync_copy(b0, table.at[pl.ds(r0, C)])

        rt = sid * RPT + (RPT // C) * C
        tail = RPT - (RPT // C) * C
        pltpu.sync_copy(h_hbm.at[pl.ds(rt, tail), pl.ds(cid * FW, FW)],
                        b0.at[pl.ds(0, tail)])
        pltpu.sync_copy(b0.at[pl.ds(0, tail)], table.at[pl.ds(rt, tail)])

        @pl.loop(0, ZR)
        def _(i):
            @pl.loop(0, FW // 16)
            def _(j):
                zbuf[i, pl.ds(j * 16, 16)] = jnp.zeros((16,), jnp.float32)

        @pl.loop(0, RPT // ZR)
        def _(k):
            pltpu.sync_copy(zbuf, acc.at[pl.ds(sid * RPT + k * ZR, ZR)])

        pltpu.sync_copy(e_hbm.at[0, pl.ds(sid * EW, EW)], src_v)
        pltpu.sync_copy(e_hbm.at[1, pl.ds(sid * EW, EW)], dst_v)

        plsc.subcore_barrier()

        g_issue(0, 0)
        g_issue(1, 1)
        g_wait(0, 0)
        s_issue(0, 0)
        g_issue(2, 2)
        g_wait(1, 1)
        s_issue(1, 1)
        g_issue(3, 3)

        @pl.loop(0, (KC - 6) // 4)
        def _(k):
            i = 4 * k
            for off in (2, 3, 4, 5):
                bb = off % 4
                g_wait(i + off, bb)
                s_issue(i + off, bb)
                s_wait(i + off - 2, (off - 2) % 4)
                g_issue(i + off + 2, (off + 2) % 4)

        g_wait(KC - 4, 2)
        s_issue(KC - 4, 2)
        s_wait(KC - 6, 0)
        g_issue(KC - 2, 0)
        g_wait(KC - 3, 3)
        s_issue(KC - 3, 3)
        s_wait(KC - 5, 1)
        g_issue(KC - 1, 1)

        g_wait(KC - 2, 0)
        s_issue(KC - 2, 0)
        s_wait(KC - 4, 2)
        g_wait(KC - 1, 1)
        s_issue(KC - 1, 1)
        s_wait(KC - 3, 3)
        s_wait(KC - 2, 0)
        s_wait(KC - 1, 1)

        plsc.subcore_barrier()

        pltpu.sync_copy(acc.at[pl.ds(sid * RPT, RPT)],
                        out_hbm.at[pl.ds(sid * RPT, RPT), pl.ds(cid * FW, FW)])

    return scatter_k


_scatter = _make_scatter()

_BN = 1000


def _combine_body(p_ref, h_ref, w_ref, wr_ref, b_ref, o_ref):
    acc = jnp.dot(p_ref[...], w_ref[...], preferred_element_type=jnp.float32)
    acc += jnp.dot(h_ref[...], wr_ref[...], preferred_element_type=jnp.float32)
    o_ref[...] = jnp.maximum(acc + b_ref[...], 0.0)


def _combine(p, h, W, b2, Wr):
    return pl.pallas_call(
        _combine_body,
        grid=(N // _BN,),
        in_specs=[
            pl.BlockSpec((_BN, H), lambda i: (i, 0)),
            pl.BlockSpec((_BN, H), lambda i: (i, 0)),
            pl.BlockSpec((H, H), lambda i: (0, 0)),
            pl.BlockSpec((H, H), lambda i: (0, 0)),
            pl.BlockSpec((1, H), lambda i: (0, 0)),
        ],
        out_specs=pl.BlockSpec((_BN, H), lambda i: (i, 0)),
        out_shape=jax.ShapeDtypeStruct((N, H), jnp.float32),
    )(p, h, W, Wr, b2)


def kernel(x, edge_index, W0, b0, Wr0, W1, b1, Wr1):
    b0r = b0.reshape(1, H)
    b1r = b1.reshape(1, H)

    p0, _ = _scatter(x, edge_index)
    h1 = _combine(p0, x, W0, b0r, Wr0)
    p1, _ = _scatter(h1, edge_index)
    h2 = _combine(p1, h1, W1, b1r, Wr1)
    return h2


# speedup vs baseline: 12.4671x; 1.1586x over previous
import functools

import jax
import jax.numpy as jnp
from jax import lax
from jax.experimental import pallas as pl
from jax.experimental.pallas import tpu as pltpu
from jax.experimental.pallas import tpu_sc as plsc

N = 10000
H = 128
FW = 64
E = 320000
NC = 2
NS = 16
C = 128
EW = E // NS
KCF = EW // C
TAIL = EW - KCF * C
RPT = N // NS


def _make_scatter():
    mesh = plsc.VectorSubcoreMesh(core_axis_name="c", subcore_axis_name="s")

    @functools.partial(
        pl.kernel,
        out_type=(
            jax.ShapeDtypeStruct((N, H), jnp.float32),
            jax.ShapeDtypeStruct((NC, N, FW), jnp.float32),
        ),
        mesh=mesh,
        scratch_types=[
            pltpu.VMEM((EW,), jnp.int32),
            pltpu.VMEM((EW,), jnp.int32),
            pltpu.VMEM((C, FW), jnp.float32),
            pltpu.VMEM((C, FW), jnp.float32),
            pltpu.VMEM((C, FW), jnp.float32),
            pltpu.VMEM((C, FW), jnp.float32),
            pltpu.VMEM_SHARED((N, FW), jnp.float32),
            pltpu.SemaphoreType.DMA,
            pltpu.SemaphoreType.DMA,
            pltpu.SemaphoreType.DMA,
            pltpu.SemaphoreType.DMA,
            pltpu.SemaphoreType.DMA,
            pltpu.SemaphoreType.DMA,
            pltpu.SemaphoreType.DMA,
            pltpu.SemaphoreType.DMA,
        ],
        compiler_params=pltpu.CompilerParams(use_tc_tiling_on_sc=False),
    )
    def scatter_k(h_hbm, e_hbm, out_hbm, tab_hbm,
                  src_v, dst_v, b0, b1, b2, b3, acc,
                  g0, g1, g2, g3, s0, s1, s2, s3):
        cid = lax.axis_index("c")
        sid = lax.axis_index("s")
        table = tab_hbm.at[cid]
        bufs = (b0, b1, b2, b3)
        gsems = (g0, g1, g2, g3)
        ssems = (s0, s1, s2, s3)

        def src_idx(i):
            return src_v.at[pl.ds(i * C, C)]

        def dst_idx(i):
            return dst_v.at[pl.ds(i * C, C)]

        def g_issue(i, b):
            pltpu.async_copy(table.at[src_idx(i)], bufs[b], gsems[b])

        def g_wait(i, b):
            pltpu.make_async_copy(table.at[src_idx(i)], bufs[b], gsems[b]).wait()

        def s_issue(i, b):
            pltpu.async_copy(bufs[b], acc.at[dst_idx(i)], ssems[b], add=True)

        def s_wait(i, b):
            pltpu.make_async_copy(bufs[b], acc.at[dst_idx(i)], ssems[b]).wait()

        r0 = sid * RPT

        idx_s = pltpu.async_copy(e_hbm.at[0, pl.ds(sid * EW, EW)], src_v, g0)
        idx_d = pltpu.async_copy(e_hbm.at[1, pl.ds(sid * EW, EW)], dst_v, g1)

        for bb in (b1, b2, b3):
            @pl.loop(0, C)
            def _(i):
                @pl.loop(0, FW // 16)
                def _(j):
                    bb[i, pl.ds(j * 16, 16)] = jnp.zeros((16,), jnp.float32)

        z1 = pltpu.async_copy(b1, acc.at[pl.ds(r0, C)], s1)
        z2 = pltpu.async_copy(b2, acc.at[pl.ds(r0 + C, C)], s2)
        z3 = pltpu.async_copy(b3, acc.at[pl.ds(r0 + 2 * C, C)], s3)
        z1.wait()
        z4 = pltpu.async_copy(b1, acc.at[pl.ds(r0 + 3 * C, C)], s1)
        z4.wait()
        z5 = pltpu.async_copy(
            b1.at[pl.ds(0, RPT - 4 * C)],
            acc.at[pl.ds(r0 + 4 * C, RPT - 4 * C)], s1)

        @pl.loop(0, RPT // C)
        def _(j):
            rr = r0 + j * C
            pltpu.sync_copy(h_hbm.at[pl.ds(rr, C), pl.ds(cid * FW, FW)], b0)
            pltpu.sync_copy(b0, table.at[pl.ds(rr, C)])

        rt = r0 + (RPT // C) * C
        tl = RPT - (RPT // C) * C
        pltpu.sync_copy(h_hbm.at[pl.ds(rt, tl), pl.ds(cid * FW, FW)],
                        b0.at[pl.ds(0, tl)])
        pltpu.sync_copy(b0.at[pl.ds(0, tl)], table.at[pl.ds(rt, tl)])

        idx_s.wait()
        idx_d.wait()
        z5.wait()
        z2.wait()
        z3.wait()

        plsc.subcore_barrier()

        g_issue(0, 0)
        g_issue(1, 1)
        g_wait(0, 0)
        s_issue(0, 0)
        g_issue(2, 2)
        g_wait(1, 1)
        s_issue(1, 1)
        g_issue(3, 3)

        @pl.loop(0, (KCF - 4) // 4)
        def _(k):
            i = 4 * k
            for off in (2, 3, 4, 5):
                bb = off % 4
                g_wait(i + off, bb)
                s_issue(i + off, bb)
                s_wait(i + off - 2, (off - 2) % 4)
                g_issue(i + off + 2, (off + 2) % 4)

        g_wait(KCF - 2, 2)
        s_issue(KCF - 2, 2)
        s_wait(KCF - 4, 0)
        g_wait(KCF - 1, 3)
        s_issue(KCF - 1, 3)
        s_wait(KCF - 3, 1)
        s_wait(KCF - 2, 2)
        s_wait(KCF - 1, 3)

        t0 = KCF * C
        pltpu.async_copy(
            table.at[src_v.at[pl.ds(t0, TAIL)]], b0.at[pl.ds(0, TAIL)], g0
        ).wait()
        pltpu.async_copy(
            b0.at[pl.ds(0, TAIL)], acc.at[dst_v.at[pl.ds(t0, TAIL)]], s0,
            add=True,
        ).wait()

        plsc.subcore_barrier()

        pltpu.sync_copy(acc.at[pl.ds(r0, RPT)],
                        out_hbm.at[pl.ds(r0, RPT), pl.ds(cid * FW, FW)])

    return scatter_k


_scatter = _make_scatter()

_BN = 1000


def _combine_body(p_ref, h_ref, w_ref, wr_ref, b_ref, o_ref):
    acc = jnp.dot(p_ref[...], w_ref[...], preferred_element_type=jnp.float32)
    acc += jnp.dot(h_ref[...], wr_ref[...], preferred_element_type=jnp.float32)
    o_ref[...] = jnp.maximum(acc + b_ref[...], 0.0)


def _combine(p, h, W, b2, Wr):
    return pl.pallas_call(
        _combine_body,
        grid=(N // _BN,),
        in_specs=[
            pl.BlockSpec((_BN, H), lambda i: (i, 0)),
            pl.BlockSpec((_BN, H), lambda i: (i, 0)),
            pl.BlockSpec((H, H), lambda i: (0, 0)),
            pl.BlockSpec((H, H), lambda i: (0, 0)),
            pl.BlockSpec((1, H), lambda i: (0, 0)),
        ],
        out_specs=pl.BlockSpec((_BN, H), lambda i: (i, 0)),
        out_shape=jax.ShapeDtypeStruct((N, H), jnp.float32),
    )(p, h, W, Wr, b2)


def kernel(x, edge_index, W0, b0, Wr0, W1, b1, Wr1):
    b0r = b0.reshape(1, H)
    b1r = b1.reshape(1, H)

    p0, _ = _scatter(x, edge_index)
    h1 = _combine(p0, x, W0, b0r, Wr0)
    p1, _ = _scatter(h1, edge_index)
    h2 = _combine(p1, h1, W1, b1r, Wr1)
    return h2


# speedup vs baseline: 12.7174x; 1.0201x over previous
import functools

import jax
import jax.numpy as jnp
from jax import lax
from jax.experimental import pallas as pl
from jax.experimental.pallas import tpu as pltpu
from jax.experimental.pallas import tpu_sc as plsc

N = 10000
H = 128
FW = 64
E = 320000
NC = 2
NS = 16
C = 128
EW = E // NS
KCF = EW // C
TAIL = EW - KCF * C
RPT = N // NS


def _make_scatter():
    mesh = plsc.VectorSubcoreMesh(core_axis_name="c", subcore_axis_name="s")

    @functools.partial(
        pl.kernel,
        out_type=(
            jax.ShapeDtypeStruct((N, H), jnp.float32),
            jax.ShapeDtypeStruct((NC, N, FW), jnp.float32),
        ),
        mesh=mesh,
        scratch_types=[
            pltpu.VMEM((EW,), jnp.int32),
            pltpu.VMEM((EW,), jnp.int32),
            pltpu.VMEM((C, FW), jnp.float32),
            pltpu.VMEM((C, FW), jnp.float32),
            pltpu.VMEM((C, FW), jnp.float32),
            pltpu.VMEM((C, FW), jnp.float32),
            pltpu.VMEM_SHARED((N, FW), jnp.float32),
            pltpu.SemaphoreType.DMA,
            pltpu.SemaphoreType.DMA,
            pltpu.SemaphoreType.DMA,
            pltpu.SemaphoreType.DMA,
            pltpu.SemaphoreType.DMA,
            pltpu.SemaphoreType.DMA,
            pltpu.SemaphoreType.DMA,
            pltpu.SemaphoreType.DMA,
        ],
        compiler_params=pltpu.CompilerParams(use_tc_tiling_on_sc=False),
    )
    def scatter_k(h_hbm, e_hbm, out_hbm, tab_hbm,
                  src_v, dst_v, b0, b1, b2, b3, acc,
                  g0, g1, g2, g3, s0, s1, s2, s3):
        cid = lax.axis_index("c")
        sid = lax.axis_index("s")
        table = tab_hbm.at[cid]
        bufs = (b0, b1, b2, b3)
        gsems = (g0, g1, g2, g3)
        ssems = (s0, s1, s2, s3)

        def src_idx(i):
            return src_v.at[pl.ds(i * C, C)]

        def dst_idx(i):
            return dst_v.at[pl.ds(i * C, C)]

        def g_issue(i, b):
            pltpu.async_copy(table.at[src_idx(i)], bufs[b], gsems[b])

        def g_wait(i, b):
            pltpu.make_async_copy(table.at[src_idx(i)], bufs[b], gsems[b]).wait()

        def s_issue(i, b):
            pltpu.async_copy(bufs[b], acc.at[dst_idx(i)], ssems[b], add=True)

        def s_wait(i, b):
            pltpu.make_async_copy(bufs[b], acc.at[dst_idx(i)], ssems[b]).wait()

        r0 = sid * RPT

        idx_s = pltpu.async_copy(e_hbm.at[0, pl.ds(sid * EW, EW)], src_v, g0)
        idx_d = pltpu.async_copy(e_hbm.at[1, pl.ds(sid * EW, EW)], dst_v, g1)

        @pl.loop(0, C)
        def _(i):
            @pl.loop(0, FW // 16)
            def _(j):
                b1[i, pl.ds(j * 16, 16)] = jnp.zeros((16,), jnp.float32)

        def rp_rows(j):
            rr = r0 + j * C
            nr = C if j < RPT // C else RPT - (RPT // C) * C
            return rr, nr

        def rp_read(j, bb, sem):
            rr, nr = rp_rows(j)
            return pltpu.async_copy(
                h_hbm.at[pl.ds(rr, nr), pl.ds(cid * FW, FW)],
                bb.at[pl.ds(0, nr)], sem)

        def rp_write(j, bb, sem):
            rr, nr = rp_rows(j)
            return pltpu.async_copy(bb.at[pl.ds(0, nr)],
                                    table.at[pl.ds(rr, nr)], sem)

        def z_issue(k):
            if k < 4:
                return pltpu.async_copy(b1, acc.at[pl.ds(r0 + k * C, C)], s1)
            return pltpu.async_copy(
                b1.at[pl.ds(0, RPT - 4 * C)],
                acc.at[pl.ds(r0 + 4 * C, RPT - 4 * C)], s1)

        z = z_issue(0)
        rd0 = rp_read(0, b0, g2)
        rd1 = rp_read(1, b2, g3)
        rd0.wait()
        wr0 = rp_write(0, b0, s0)
        rd1.wait()
        wr1 = rp_write(1, b2, s2)
        z.wait()
        z = z_issue(1)
        wr0.wait()
        rd2 = rp_read(2, b0, g2)
        wr1.wait()
        rd3 = rp_read(3, b2, g3)
        z.wait()
        z = z_issue(2)
        rd2.wait()
        wr2 = rp_write(2, b0, s0)
        rd3.wait()
        wr3 = rp_write(3, b2, s2)
        z.wait()
        z = z_issue(3)
        wr2.wait()
        rd4 = rp_read(4, b0, g2)
        z.wait()
        z = z_issue(4)
        rd4.wait()
        wr3.wait()
        wr4 = rp_write(4, b0, s0)
        wr4.wait()
        z.wait()

        idx_s.wait()
        idx_d.wait()

        plsc.subcore_barrier()

        g_issue(0, 0)
        g_issue(1, 1)
        g_wait(0, 0)
        s_issue(0, 0)
        g_issue(2, 2)
        g_wait(1, 1)
        s_issue(1, 1)
        g_issue(3, 3)

        @pl.loop(0, (KCF - 4) // 4)
        def _(k):
            i = 4 * k
            for off in (2, 3, 4, 5):
                bb = off % 4
                g_wait(i + off, bb)
                s_issue(i + off, bb)
                s_wait(i + off - 2, (off - 2) % 4)
                g_issue(i + off + 2, (off + 2) % 4)

        g_wait(KCF - 2, 2)
        s_issue(KCF - 2, 2)
        s_wait(KCF - 4, 0)
        g_wait(KCF - 1, 3)
        s_issue(KCF - 1, 3)
        s_wait(KCF - 3, 1)
        s_wait(KCF - 2, 2)
        s_wait(KCF - 1, 3)

        t0 = KCF * C
        pltpu.async_copy(
            table.at[src_v.at[pl.ds(t0, TAIL)]], b0.at[pl.ds(0, TAIL)], g0
        ).wait()
        pltpu.async_copy(
            b0.at[pl.ds(0, TAIL)], acc.at[dst_v.at[pl.ds(t0, TAIL)]], s0,
            add=True,
        ).wait()

        plsc.subcore_barrier()

        pltpu.sync_copy(acc.at[pl.ds(r0, RPT)],
                        out_hbm.at[pl.ds(r0, RPT), pl.ds(cid * FW, FW)])

    return scatter_k


_scatter = _make_scatter()

_BN = 1000


def _combine_body(p_ref, h_ref, w_ref, wr_ref, b_ref, o_ref):
    acc = jnp.dot(p_ref[...], w_ref[...], preferred_element_type=jnp.float32)
    acc += jnp.dot(h_ref[...], wr_ref[...], preferred_element_type=jnp.float32)
    o_ref[...] = jnp.maximum(acc + b_ref[...], 0.0)


def _combine(p, h, W, b2, Wr):
    return pl.pallas_call(
        _combine_body,
        grid=(N // _BN,),
        in_specs=[
            pl.BlockSpec((_BN, H), lambda i: (i, 0)),
            pl.BlockSpec((_BN, H), lambda i: (i, 0)),
            pl.BlockSpec((H, H), lambda i: (0, 0)),
            pl.BlockSpec((H, H), lambda i: (0, 0)),
            pl.BlockSpec((1, H), lambda i: (0, 0)),
        ],
        out_specs=pl.BlockSpec((_BN, H), lambda i: (i, 0)),
        out_shape=jax.ShapeDtypeStruct((N, H), jnp.float32),
    )(p, h, W, Wr, b2)


def kernel(x, edge_index, W0, b0, Wr0, W1, b1, Wr1):
    b0r = b0.reshape(1, H)
    b1r = b1.reshape(1, H)

    p0, _ = _scatter(x, edge_index)
    h1 = _combine(p0, x, W0, b0r, Wr0)
    p1, _ = _scatter(h1, edge_index)
    h2 = _combine(p1, h1, W1, b1r, Wr1)
    return h2


# speedup vs baseline: 15.4828x; 1.2175x over previous
import functools

import jax
import jax.numpy as jnp
from jax import lax
from jax.experimental import pallas as pl
from jax.experimental.pallas import tpu as pltpu
from jax.experimental.pallas import tpu_sc as plsc

N = 10000
H = 128
FW = 64
E = 320000
NC = 2
NS = 16
C = 128
EW = E // NS
KCF = EW // C
TAIL = EW - KCF * C
RPT = N // NS


def _make_scatter():
    mesh = plsc.VectorSubcoreMesh(core_axis_name="c", subcore_axis_name="s")

    @functools.partial(
        pl.kernel,
        out_type=(
            jax.ShapeDtypeStruct((N, H), jnp.float32),
            jax.ShapeDtypeStruct((NC, N, FW), jnp.float32),
        ),
        mesh=mesh,
        scratch_types=[
            pltpu.VMEM((EW,), jnp.int32),
            pltpu.VMEM((EW,), jnp.int32),
            pltpu.VMEM((C, FW), jnp.float32),
            pltpu.VMEM((C, FW), jnp.float32),
            pltpu.VMEM((C, FW), jnp.float32),
            pltpu.VMEM((C, FW), jnp.float32),
            pltpu.VMEM((C, FW), jnp.float32),
            pltpu.VMEM((C, FW), jnp.float32),
            pltpu.VMEM_SHARED((N, FW), jnp.float32),
            pltpu.SemaphoreType.DMA,
            pltpu.SemaphoreType.DMA,
            pltpu.SemaphoreType.DMA,
            pltpu.SemaphoreType.DMA,
            pltpu.SemaphoreType.DMA,
            pltpu.SemaphoreType.DMA,
            pltpu.SemaphoreType.DMA,
            pltpu.SemaphoreType.DMA,
            pltpu.SemaphoreType.DMA,
            pltpu.SemaphoreType.DMA,
            pltpu.SemaphoreType.DMA,
            pltpu.SemaphoreType.DMA,
        ],
        compiler_params=pltpu.CompilerParams(use_tc_tiling_on_sc=False),
    )
    def scatter_k(h_hbm, e_hbm, out_hbm, tab_hbm,
                  src_v, dst_v, b0, b1, b2, b3, b4, b5, acc,
                  g0, g1, g2, g3, g4, g5, s0, s1, s2, s3, s4, s5):
        cid = lax.axis_index("c")
        sid = lax.axis_index("s")
        table = tab_hbm.at[cid]
        bufs = (b0, b1, b2, b3, b4, b5)
        gsems = (g0, g1, g2, g3, g4, g5)
        ssems = (s0, s1, s2, s3, s4, s5)

        def src_idx(i):
            return src_v.at[pl.ds(i * C, C)]

        def dst_idx(i):
            return dst_v.at[pl.ds(i * C, C)]

        def g_issue(i, b):
            pltpu.async_copy(table.at[src_idx(i)], bufs[b], gsems[b])

        def g_wait(i, b):
            pltpu.make_async_copy(table.at[src_idx(i)], bufs[b], gsems[b]).wait()

        def s_issue(i, b):
            pltpu.async_copy(bufs[b], acc.at[dst_idx(i)], ssems[b], add=True)

        def s_wait(i, b):
            pltpu.make_async_copy(bufs[b], acc.at[dst_idx(i)], ssems[b]).wait()

        r0 = sid * RPT

        idx_s = pltpu.async_copy(e_hbm.at[0, pl.ds(sid * EW, EW)], src_v, g0)
        idx_d = pltpu.async_copy(e_hbm.at[1, pl.ds(sid * EW, EW)], dst_v, g1)

        @pl.loop(0, C)
        def _(i):
            @pl.loop(0, FW // 16)
            def _(j):
                b1[i, pl.ds(j * 16, 16)] = jnp.zeros((16,), jnp.float32)

        def rp_rows(j):
            rr = r0 + j * C
            nr = C if j < RPT // C else RPT - (RPT // C) * C
            return rr, nr

        def rp_read(j, bb, sem):
            rr, nr = rp_rows(j)
            return pltpu.async_copy(
                h_hbm.at[pl.ds(rr, nr), pl.ds(cid * FW, FW)],
                bb.at[pl.ds(0, nr)], sem)

        def rp_write(j, bb, sem):
            rr, nr = rp_rows(j)
            return pltpu.async_copy(bb.at[pl.ds(0, nr)],
                                    table.at[pl.ds(rr, nr)], sem)

        def z_issue(k):
            if k < 4:
                return pltpu.async_copy(b1, acc.at[pl.ds(r0 + k * C, C)], s1)
            return pltpu.async_copy(
                b1.at[pl.ds(0, RPT - 4 * C)],
                acc.at[pl.ds(r0 + 4 * C, RPT - 4 * C)], s1)

        z = z_issue(0)
        rd0 = rp_read(0, b0, g2)
        rd1 = rp_read(1, b2, g3)
        rd0.wait()
        wr0 = rp_write(0, b0, s0)
        rd1.wait()
        wr1 = rp_write(1, b2, s2)
        z.wait()
        z = z_issue(1)
        wr0.wait()
        rd2 = rp_read(2, b0, g2)
        wr1.wait()
        rd3 = rp_read(3, b2, g3)
        z.wait()
        z = z_issue(2)
        rd2.wait()
        wr2 = rp_write(2, b0, s0)
        rd3.wait()
        wr3 = rp_write(3, b2, s2)
        z.wait()
        z = z_issue(3)
        wr2.wait()
        rd4 = rp_read(4, b0, g2)
        z.wait()
        z = z_issue(4)
        rd4.wait()
        wr3.wait()
        wr4 = rp_write(4, b0, s0)
        wr4.wait()
        z.wait()

        idx_s.wait()
        idx_d.wait()

        plsc.subcore_barrier()

        for i in range(4):
            g_issue(i, i)
        g_wait(0, 0)
        s_issue(0, 0)
        g_issue(4, 4)
        g_wait(1, 1)
        s_issue(1, 1)
        g_issue(5, 5)

        @pl.loop(0, (KCF - 6) // 6)
        def _(k):
            i = 6 * k
            for off in (2, 3, 4, 5, 6, 7):
                bb = off % 6
                g_wait(i + off, bb)
                s_issue(i + off, bb)
                s_wait(i + off - 2, (off - 2) % 6)
                g_issue(i + off + 4, (off + 4) % 6)

        for i in range(KCF - 4, KCF):
            bb = i % 6
            g_wait(i, bb)
            s_issue(i, bb)
            s_wait(i - 2, (i - 2) % 6)
        s_wait(KCF - 2, (KCF - 2) % 6)
        s_wait(KCF - 1, (KCF - 1) % 6)

        t0 = KCF * C
        pltpu.async_copy(
            table.at[src_v.at[pl.ds(t0, TAIL)]], b0.at[pl.ds(0, TAIL)], g0
        ).wait()
        pltpu.async_copy(
            b0.at[pl.ds(0, TAIL)], acc.at[dst_v.at[pl.ds(t0, TAIL)]], s0,
            add=True,
        ).wait()

        plsc.subcore_barrier()

        pltpu.sync_copy(acc.at[pl.ds(r0, RPT)],
                        out_hbm.at[pl.ds(r0, RPT), pl.ds(cid * FW, FW)])

    return scatter_k


_scatter = _make_scatter()

_BN = 1000


def _combine_body(p_ref, h_ref, w_ref, wr_ref, b_ref, o_ref):
    acc = jnp.dot(p_ref[...], w_ref[...], preferred_element_type=jnp.float32)
    acc += jnp.dot(h_ref[...], wr_ref[...], preferred_element_type=jnp.float32)
    o_ref[...] = jnp.maximum(acc + b_ref[...], 0.0)


def _combine(p, h, W, b2, Wr):
    return pl.pallas_call(
        _combine_body,
        grid=(N // _BN,),
        in_specs=[
            pl.BlockSpec((_BN, H), lambda i: (i, 0)),
            pl.BlockSpec((_BN, H), lambda i: (i, 0)),
            pl.BlockSpec((H, H), lambda i: (0, 0)),
            pl.BlockSpec((H, H), lambda i: (0, 0)),
            pl.BlockSpec((1, H), lambda i: (0, 0)),
        ],
        out_specs=pl.BlockSpec((_BN, H), lambda i: (i, 0)),
        out_shape=jax.ShapeDtypeStruct((N, H), jnp.float32),
    )(p, h, W, Wr, b2)


def kernel(x, edge_index, W0, b0, Wr0, W1, b1, Wr1):
    b0r = b0.reshape(1, H)
    b1r = b1.reshape(1, H)

    p0, _ = _scatter(x, edge_index)
    h1 = _combine(p0, x, W0, b0r, Wr0)
    p1, _ = _scatter(h1, edge_index)
    h2 = _combine(p1, h1, W1, b1r, Wr1)
    return h2
